# baseline (speedup 1.0000x reference)
import jax
import jax.numpy as jnp
from jax.experimental import pallas as pl
from jax.experimental.pallas import tpu as pltpu

B = 1024
K = 2048
N = 100000
BN = 1000
NBLK = N // BN
NSTEP = NBLK // 4


LR0 = (BN * 3 // 4 // 8) * 8
LR1 = BN - LR0


def _body(x_hbm, lut_hbm, out_hbm, x_v, l0, l1, l2, l3, o0, o1,
          sx, sl0, sl1, sl2, sl3, st0, st1, st2, st3, so0, so1):
    lbufs = (l0, l1, l2, l3)
    lsems = (sl0, sl1, sl2, sl3)
    tsems = (st0, st1, st2, st3)

    def lut_in_a(j, c):
        return pltpu.make_async_copy(
            lut_hbm.at[pl.ds(j * BN, LR0), :],
            lbufs[c].at[pl.ds(0, LR0), :], lsems[c])

    def lut_in_b(j, c):
        return pltpu.make_async_copy(
            lut_hbm.at[pl.ds(j * BN + LR0, LR1), :],
            lbufs[c].at[pl.ds(LR0, LR1), :], tsems[c])

    class _Pair:
        def __init__(self, j, c):
            self.a = lut_in_a(j, c)
            self.b = lut_in_b(j, c)

        def start(self):
            self.a.start(priority=0)
            self.b.start(priority=1)

        def wait(self):
            self.a.wait()
            self.b.wait()

    def lut_in(j, c):
        return _Pair(j, c)

    def out_w(j, buf, sem):
        return pltpu.make_async_copy(
            buf, out_hbm.at[pl.ds(j * BN, BN), :], sem)

    xcopy = pltpu.make_async_copy(x_hbm, x_v, sx)
    xcopy.start()
    for c in range(4):
        lut_in(c, c).start()
    xcopy.wait()
    xb = x_v[...]

    def dot(lbuf):
        return jax.lax.dot_general(
            lbuf[...].astype(jnp.bfloat16), xb,
            dimension_numbers=(((1,), (1,)), ((), ())),
            preferred_element_type=jnp.float32)

    def step(i, carry):
        j0 = 4 * i

        @pl.when(i > 0)
        def _():
            out_w(j0 - 2, o0, so0).wait()

        lut_in(j0, 0).wait()
        o0[...] = dot(l0)
        out_w(j0, o0, so0).start(priority=1)

        @pl.when(i + 1 < NSTEP)
        def _():
            lut_in(j0 + 4, 0).start()

        @pl.when(i > 0)
        def _():
            out_w(j0 - 1, o1, so1).wait()

        lut_in(j0 + 1, 1).wait()
        o1[...] = dot(l1)
        out_w(j0 + 1, o1, so1).start(priority=1)

        @pl.when(i + 1 < NSTEP)
        def _():
            lut_in(j0 + 5, 1).start()

        out_w(j0, o0, so0).wait()
        lut_in(j0 + 2, 2).wait()
        o0[...] = dot(l2)
        out_w(j0 + 2, o0, so0).start(priority=1)

        @pl.when(i + 1 < NSTEP)
        def _():
            lut_in(j0 + 6, 2).start()

        out_w(j0 + 1, o1, so1).wait()
        lut_in(j0 + 3, 3).wait()
        o1[...] = dot(l3)
        out_w(j0 + 3, o1, so1).start(priority=1)

        @pl.when(i + 1 < NSTEP)
        def _():
            lut_in(j0 + 7, 3).start()

        @pl.when(i + 1 == NSTEP)
        def _():
            out_w(j0 + 2, o0, so0).wait()
            out_w(j0 + 3, o1, so1).wait()

        return carry

    jax.lax.fori_loop(0, NSTEP, step, 0)


def kernel(x, person_id, LUT):
    del person_id
    xb = x.astype(jnp.bfloat16)
    out_t = pl.pallas_call(
        _body,
        in_specs=[
            pl.BlockSpec(memory_space=pl.ANY),
            pl.BlockSpec(memory_space=pl.ANY),
        ],
        out_specs=pl.BlockSpec(memory_space=pl.ANY),
        out_shape=jax.ShapeDtypeStruct((N, B), jnp.float32),
        scratch_shapes=[
            pltpu.VMEM((B, K), jnp.bfloat16),
            pltpu.VMEM((BN, K), jnp.float32),
            pltpu.VMEM((BN, K), jnp.float32),
            pltpu.VMEM((BN, K), jnp.float32),
            pltpu.VMEM((BN, K), jnp.float32),
            pltpu.VMEM((BN, B), jnp.float32),
            pltpu.VMEM((BN, B), jnp.float32),
            pltpu.SemaphoreType.DMA,
            pltpu.SemaphoreType.DMA,
            pltpu.SemaphoreType.DMA,
            pltpu.SemaphoreType.DMA,
            pltpu.SemaphoreType.DMA,
            pltpu.SemaphoreType.DMA,
            pltpu.SemaphoreType.DMA,
            pltpu.SemaphoreType.DMA,
            pltpu.SemaphoreType.DMA,
            pltpu.SemaphoreType.DMA,
            pltpu.SemaphoreType.DMA,
        ],
    )(xb, LUT)
    return out_t.T
